# baseline (speedup 1.0000x reference)
import functools

import jax
import jax.numpy as jnp
import numpy as np
from jax import lax
from jax.experimental import pallas as pl
from jax.experimental.pallas import tpu as pltpu
from jax.experimental.pallas import tpu_sc as plsc

_N = 10000
_E = 320000
_H = 128
_R = 5
_NQ = 4
_QW = 32
_NS = 16
_NC = 2
_CH = 128
_SCH = 256
_EPT = 20480
_PPT = _EPT // _SCH
_EP = _NS * _EPT
_RN = _R * _N
_STRIPE = 3128
_LSTRIPE = _RN - 15 * _STRIPE
_BM = 400


def _layer_norm(x, g, b):
    mu = jnp.mean(x, axis=-1, keepdims=True)
    var = jnp.mean(jnp.square(x - mu), axis=-1, keepdims=True)
    return (x - mu) / jnp.sqrt(var + 1e-5) * g + b


def _gelu(x):
    return x * 0.5 * (1.0 + lax.erf(x * np.float32(1.0 / np.sqrt(2.0))))


def _proj_body(x_ref, w_ref, b_ref, g_ref, be_ref, o_ref):
    h = jnp.dot(x_ref[...], w_ref[...], preferred_element_type=jnp.float32)
    o_ref[...] = _gelu(_layer_norm(h + b_ref[...], g_ref[...], be_ref[...]))


def _proj(x, w, b, g, be):
    return pl.pallas_call(
        _proj_body,
        grid=(_N // _BM,),
        in_specs=[
            pl.BlockSpec((_BM, _H), lambda i: (i, 0)),
            pl.BlockSpec((_H, _H), lambda i: (0, 0)),
            pl.BlockSpec((1, _H), lambda i: (0, 0)),
            pl.BlockSpec((1, _H), lambda i: (0, 0)),
            pl.BlockSpec((1, _H), lambda i: (0, 0)),
        ],
        out_specs=pl.BlockSpec((_BM, _H), lambda i: (i, 0)),
        out_shape=jax.ShapeDtypeStruct((_N, _H), jnp.float32),
    )(x, w, b.reshape(1, -1), g.reshape(1, -1), be.reshape(1, -1))


def _mid_body(a_ref, wq_ref, b_ref, o_ref):
    acc = jnp.zeros((_BM, _H), jnp.float32)
    for q in range(_NQ):
        acc += jnp.dot(a_ref[q], wq_ref[q], preferred_element_type=jnp.float32)
    o_ref[...] = _gelu(acc + b_ref[...])


def _mid(a2, wq, b):
    rw = _R * _QW
    return pl.pallas_call(
        _mid_body,
        grid=(_N // _BM,),
        in_specs=[
            pl.BlockSpec((_NQ, _BM, rw), lambda i: (0, i, 0)),
            pl.BlockSpec((_NQ, rw, _H), lambda i: (0, 0, 0)),
            pl.BlockSpec((1, _H), lambda i: (0, 0)),
        ],
        out_specs=pl.BlockSpec((_BM, _H), lambda i: (i, 0)),
        out_shape=jax.ShapeDtypeStruct((_N, _H), jnp.float32),
    )(a2, wq, b.reshape(1, -1))


def _head_body(h_ref, w1_ref, b1_ref, g1_ref, bb1_ref, w2_ref, b2_ref,
               wt1_ref, bt1_ref, gt1_ref, bbt1_ref, wt2_ref, bt2_ref,
               cat_ref, tax_ref):
    h = h_ref[...]
    z = _gelu(_layer_norm(
        jnp.dot(h, w1_ref[...], preferred_element_type=jnp.float32) + b1_ref[...],
        g1_ref[...], bb1_ref[...]))
    cat_ref[...] = jnp.dot(z, w2_ref[...], preferred_element_type=jnp.float32) + b2_ref[...]
    t = _gelu(_layer_norm(
        jnp.dot(h, wt1_ref[...], preferred_element_type=jnp.float32) + bt1_ref[...],
        gt1_ref[...], bbt1_ref[...]))
    tax_ref[...] = jnp.dot(t, wt2_ref[...], preferred_element_type=jnp.float32) + bt2_ref[...]


def _head(h, w1, b1, g1, bb1, w2, b2, wt1, bt1, gt1, bbt1, wt2, bt2):
    h2 = 2 * _H
    out = 400
    tax = 20
    return pl.pallas_call(
        _head_body,
        grid=(_N // _BM,),
        in_specs=[
            pl.BlockSpec((_BM, _H), lambda i: (i, 0)),
            pl.BlockSpec((_H, h2), lambda i: (0, 0)),
            pl.BlockSpec((1, h2), lambda i: (0, 0)),
            pl.BlockSpec((1, h2), lambda i: (0, 0)),
            pl.BlockSpec((1, h2), lambda i: (0, 0)),
            pl.BlockSpec((h2, out), lambda i: (0, 0)),
            pl.BlockSpec((1, out), lambda i: (0, 0)),
            pl.BlockSpec((_H, h2), lambda i: (0, 0)),
            pl.BlockSpec((1, h2), lambda i: (0, 0)),
            pl.BlockSpec((1, h2), lambda i: (0, 0)),
            pl.BlockSpec((1, h2), lambda i: (0, 0)),
            pl.BlockSpec((h2, tax), lambda i: (0, 0)),
            pl.BlockSpec((1, tax), lambda i: (0, 0)),
        ],
        out_specs=[
            pl.BlockSpec((_BM, out), lambda i: (i, 0)),
            pl.BlockSpec((_BM, tax), lambda i: (i, 0)),
        ],
        out_shape=[
            jax.ShapeDtypeStruct((_N, out), jnp.float32),
            jax.ShapeDtypeStruct((_N, tax), jnp.float32),
        ],
    )(h, w1, b1.reshape(1, -1), g1.reshape(1, -1), bb1.reshape(1, -1),
      w2, b2.reshape(1, -1), wt1, bt1.reshape(1, -1), gt1.reshape(1, -1),
      bbt1.reshape(1, -1), wt2, bt2.reshape(1, -1))


def _splat16(v, i):
    return lax.gather(
        v,
        jnp.full((16, 1), i, jnp.int32),
        lax.GatherDimensionNumbers(
            offset_dims=(), collapsed_slice_dims=(0,), start_index_map=(0,)),
        (1,),
        mode=lax.GatherScatterMode.PROMISE_IN_BOUNDS)


def _sc_body(hqf, src_all, cidx_t, attr_t, aq,
             src_v, cidx_v, attr_v, rows_v, acc, gsem, ssem):
    c = lax.axis_index("c")
    s = lax.axis_index("s")

    for qi in range(2):
        q = 2 * c + qi

        z16 = jnp.zeros((16,), jnp.float32)
        for i in range(_CH):
            rows_v[0][i, 0:16] = z16
            rows_v[0][i, 16:32] = z16

        def _zero(k, carry):
            pltpu.sync_copy(rows_v[0].at[pl.ds(0, _CH)],
                            acc.at[pl.ds(s * _STRIPE + k * _CH, _CH)])
            return carry
        lax.fori_loop(0, 3072 // _CH, _zero, 0)

        @pl.when(s < _NS - 1)
        def _zero_tail():
            pltpu.sync_copy(rows_v[0].at[pl.ds(0, 56)],
                            acc.at[pl.ds(s * _STRIPE + 3072, 56)])

        @pl.when(s == _NS - 1)
        def _zero_tail2():
            pltpu.sync_copy(rows_v[0].at[pl.ds(0, 8)],
                            acc.at[pl.ds(s * _STRIPE + 3072, 8)])
        plsc.subcore_barrier()

        def _step(st, carry):
            for k in range(2):
                jp = st * 2 + k
                @pl.when(st > 0)
                def _drain():
                    pltpu.make_async_copy(
                        rows_v[k], acc.at[cidx_v[k]], ssem).wait()
                pltpu.sync_copy(src_all.at[q, s, jp], src_v[k])
                pltpu.sync_copy(cidx_t.at[s, jp], cidx_v[k])
                pltpu.sync_copy(attr_t.at[s, jp], attr_v[k])
                pltpu.async_copy(hqf.at[src_v[k]], rows_v[k], gsem)
            for k in range(2):
                pltpu.make_async_copy(
                    hqf.at[src_v[k]], rows_v[k], gsem).wait()
                for g in range(_SCH // 16):
                    a16 = attr_v[k][pl.ds(g * 16, 16)]
                    for i in range(16):
                        e = g * 16 + i
                        spl = _splat16(a16, i)
                        rows_v[k][e, 0:16] = rows_v[k][e, 0:16] * spl
                        rows_v[k][e, 16:32] = rows_v[k][e, 16:32] * spl
                pltpu.async_copy(
                    rows_v[k], acc.at[cidx_v[k]], ssem, add=True)
            return carry
        lax.fori_loop(0, _PPT // 2, _step, 0)
        for k in range(2):
            pltpu.make_async_copy(
                rows_v[k], acc.at[cidx_v[k]], ssem).wait()
        plsc.subcore_barrier()

        @pl.when(s < _NS - 1)
        def _writeout():
            pltpu.sync_copy(acc.at[pl.ds(s * _STRIPE, _STRIPE)],
                            aq.at[q, pl.ds(s * _STRIPE, _STRIPE)])

        @pl.when(s == _NS - 1)
        def _writeout_tail():
            pltpu.sync_copy(acc.at[pl.ds(s * _STRIPE, _LSTRIPE)],
                            aq.at[q, pl.ds(s * _STRIPE, _LSTRIPE)])
        plsc.subcore_barrier()


def _sc_body_wrap(hqf, src_all, cidx_t, attr_t, aq,
                  src_v0, src_v1, cidx_v0, cidx_v1, attr_v0, attr_v1,
                  rows_v0, rows_v1, acc, gsem, ssem):
    _sc_body(hqf, src_all, cidx_t, attr_t, aq,
             (src_v0, src_v1), (cidx_v0, cidx_v1), (attr_v0, attr_v1),
             (rows_v0, rows_v1), acc, gsem, ssem)


def _sc_agg(hqf, src_all, cidx_t, attr_t):
    kern = pl.kernel(
        _sc_body_wrap,
        out_type=jax.ShapeDtypeStruct((_NQ, _RN, _QW), jnp.float32),
        mesh=plsc.VectorSubcoreMesh(core_axis_name="c", subcore_axis_name="s"),
        compiler_params=pltpu.CompilerParams(use_tc_tiling_on_sc=False),
        scratch_types=[
            pltpu.VMEM((_SCH,), jnp.int32),
            pltpu.VMEM((_SCH,), jnp.int32),
            pltpu.VMEM((_SCH,), jnp.int32),
            pltpu.VMEM((_SCH,), jnp.int32),
            pltpu.VMEM((_SCH,), jnp.float32),
            pltpu.VMEM((_SCH,), jnp.float32),
            pltpu.VMEM((_SCH, _QW), jnp.float32),
            pltpu.VMEM((_SCH, _QW), jnp.float32),
            pltpu.VMEM_SHARED((_RN, _QW), jnp.float32),
            pltpu.SemaphoreType.DMA,
            pltpu.SemaphoreType.DMA,
        ],
    )
    return kern(hqf, src_all, cidx_t, attr_t)


def kernel(x, edge_index, edge_type, edge_attr, W_in, b_in, g_in, be_in,
           Wg, bg, W1, b1, g1, bb1, W2, b2, Wt1, bt1, gt1, bbt1, Wt2, bt2):
    src = edge_index[0]
    dst = edge_index[1]
    cidx = dst * _R + edge_type
    attr = edge_attr[:, 0]

    pad = _EP - _E
    srcp = jnp.pad(src, (0, pad))
    cidxp = jnp.pad(cidx, (0, pad))
    attrp = jnp.pad(attr, (0, pad))

    qoff = (jnp.arange(_NQ, dtype=jnp.int32) * _N)[:, None]
    src_all = (srcp[None, :] + qoff).reshape(_NQ, _NS, _PPT, _SCH)
    cidx_t = cidxp.reshape(_NS, _PPT, _SCH)
    attr_t = attrp.reshape(_NS, _PPT, _SCH)

    h = _proj(x, W_in, b_in, g_in, be_in)

    for l in range(2):
        hqf = h.reshape(_N, _NQ, _QW).transpose(1, 0, 2).reshape(_NQ * _N, _QW)
        a = _sc_agg(hqf, src_all, cidx_t, attr_t)
        a2 = a.reshape(_NQ, _N, _R * _QW)
        wq = jnp.stack([
            Wg[l, :, qq * _QW:(qq + 1) * _QW, :].reshape(_R * _QW, _H)
            for qq in range(_NQ)])
        h = _mid(a2, wq, bg[l])

    return _head(h, W1, b1, g1, bb1, W2, b2, Wt1, bt1, gt1, bbt1, Wt2, bt2)


# speedup vs baseline: 1.2101x; 1.2101x over previous
import functools

import jax
import jax.numpy as jnp
import numpy as np
from jax import lax
from jax.experimental import pallas as pl
from jax.experimental.pallas import tpu as pltpu
from jax.experimental.pallas import tpu_sc as plsc

_N = 10000
_E = 320000
_H = 128
_R = 5
_NQ = 8
_QW = 16
_NS = 16
_NC = 2
_SCH = 512
_EPT = 20480
_PPT = _EPT // _SCH
_EP = _NS * _EPT
_RN = _R * _N
_STRIPE = 3128
_LSTRIPE = _RN - 15 * _STRIPE
_BM = 400


def _layer_norm(x, g, b):
    mu = jnp.mean(x, axis=-1, keepdims=True)
    var = jnp.mean(jnp.square(x - mu), axis=-1, keepdims=True)
    return (x - mu) / jnp.sqrt(var + 1e-5) * g + b


def _gelu(x):
    return x * 0.5 * (1.0 + lax.erf(x * np.float32(1.0 / np.sqrt(2.0))))


def _proj_body(x_ref, w_ref, b_ref, g_ref, be_ref, o_ref):
    h = jnp.dot(x_ref[...], w_ref[...], preferred_element_type=jnp.float32)
    o_ref[...] = _gelu(_layer_norm(h + b_ref[...], g_ref[...], be_ref[...]))


def _proj(x, w, b, g, be):
    return pl.pallas_call(
        _proj_body,
        grid=(_N // _BM,),
        in_specs=[
            pl.BlockSpec((_BM, _H), lambda i: (i, 0)),
            pl.BlockSpec((_H, _H), lambda i: (0, 0)),
            pl.BlockSpec((1, _H), lambda i: (0, 0)),
            pl.BlockSpec((1, _H), lambda i: (0, 0)),
            pl.BlockSpec((1, _H), lambda i: (0, 0)),
        ],
        out_specs=pl.BlockSpec((_BM, _H), lambda i: (i, 0)),
        out_shape=jax.ShapeDtypeStruct((_N, _H), jnp.float32),
    )(x, w, b.reshape(1, -1), g.reshape(1, -1), be.reshape(1, -1))


def _mid_body(a_ref, wq_ref, b_ref, o_ref):
    acc = jnp.zeros((_BM, _H), jnp.float32)
    for q in range(_NQ):
        acc += jnp.dot(a_ref[q], wq_ref[q], preferred_element_type=jnp.float32)
    o_ref[...] = _gelu(acc + b_ref[...])


def _mid(a2, wq, b):
    rw = _R * _QW
    return pl.pallas_call(
        _mid_body,
        grid=(_N // _BM,),
        in_specs=[
            pl.BlockSpec((_NQ, _BM, rw), lambda i: (0, i, 0)),
            pl.BlockSpec((_NQ, rw, _H), lambda i: (0, 0, 0)),
            pl.BlockSpec((1, _H), lambda i: (0, 0)),
        ],
        out_specs=pl.BlockSpec((_BM, _H), lambda i: (i, 0)),
        out_shape=jax.ShapeDtypeStruct((_N, _H), jnp.float32),
    )(a2, wq, b.reshape(1, -1))


def _head_body(h_ref, w1_ref, b1_ref, g1_ref, bb1_ref, w2_ref, b2_ref,
               wt1_ref, bt1_ref, gt1_ref, bbt1_ref, wt2_ref, bt2_ref,
               cat_ref, tax_ref):
    h = h_ref[...]
    z = _gelu(_layer_norm(
        jnp.dot(h, w1_ref[...], preferred_element_type=jnp.float32) + b1_ref[...],
        g1_ref[...], bb1_ref[...]))
    cat_ref[...] = jnp.dot(z, w2_ref[...], preferred_element_type=jnp.float32) + b2_ref[...]
    t = _gelu(_layer_norm(
        jnp.dot(h, wt1_ref[...], preferred_element_type=jnp.float32) + bt1_ref[...],
        gt1_ref[...], bbt1_ref[...]))
    tax_ref[...] = jnp.dot(t, wt2_ref[...], preferred_element_type=jnp.float32) + bt2_ref[...]


def _head(h, w1, b1, g1, bb1, w2, b2, wt1, bt1, gt1, bbt1, wt2, bt2):
    h2 = 2 * _H
    out = 400
    tax = 20
    return pl.pallas_call(
        _head_body,
        grid=(_N // _BM,),
        in_specs=[
            pl.BlockSpec((_BM, _H), lambda i: (i, 0)),
            pl.BlockSpec((_H, h2), lambda i: (0, 0)),
            pl.BlockSpec((1, h2), lambda i: (0, 0)),
            pl.BlockSpec((1, h2), lambda i: (0, 0)),
            pl.BlockSpec((1, h2), lambda i: (0, 0)),
            pl.BlockSpec((h2, out), lambda i: (0, 0)),
            pl.BlockSpec((1, out), lambda i: (0, 0)),
            pl.BlockSpec((_H, h2), lambda i: (0, 0)),
            pl.BlockSpec((1, h2), lambda i: (0, 0)),
            pl.BlockSpec((1, h2), lambda i: (0, 0)),
            pl.BlockSpec((1, h2), lambda i: (0, 0)),
            pl.BlockSpec((h2, tax), lambda i: (0, 0)),
            pl.BlockSpec((1, tax), lambda i: (0, 0)),
        ],
        out_specs=[
            pl.BlockSpec((_BM, out), lambda i: (i, 0)),
            pl.BlockSpec((_BM, tax), lambda i: (i, 0)),
        ],
        out_shape=[
            jax.ShapeDtypeStruct((_N, out), jnp.float32),
            jax.ShapeDtypeStruct((_N, tax), jnp.float32),
        ],
    )(h, w1, b1.reshape(1, -1), g1.reshape(1, -1), bb1.reshape(1, -1),
      w2, b2.reshape(1, -1), wt1, bt1.reshape(1, -1), gt1.reshape(1, -1),
      bbt1.reshape(1, -1), wt2, bt2.reshape(1, -1))


def _splat16(v, i):
    return lax.gather(
        v,
        jnp.full((16, 1), i, jnp.int32),
        lax.GatherDimensionNumbers(
            offset_dims=(), collapsed_slice_dims=(0,), start_index_map=(0,)),
        (1,),
        mode=lax.GatherScatterMode.PROMISE_IN_BOUNDS)


def _sc_body(hq, src_t, cidx_t, attr_t, aq,
             src_v, cidx_v, attr_v, rows_v, acc, htab, gsem, ssem):
    c = lax.axis_index("c")
    s = lax.axis_index("s")

    for qi in range(_NQ // _NC):
        q = (_NQ // _NC) * c + qi

        z16 = jnp.zeros((16,), jnp.float32)

        def _zfill(jj, carry):
            for i2 in range(16):
                rows_v[0][jj * 16 + i2, 0:16] = z16
            return carry
        lax.fori_loop(0, _SCH // 16, _zfill, 0)

        @pl.when(s == 0)
        def _load_table():
            pltpu.sync_copy(hq.at[q], htab)

        def _zero(k, carry):
            pltpu.sync_copy(rows_v[0],
                            acc.at[pl.ds(s * _STRIPE + k * _SCH, _SCH)])
            return carry
        lax.fori_loop(0, 3072 // _SCH, _zero, 0)

        @pl.when(s < _NS - 1)
        def _zero_tail():
            pltpu.sync_copy(rows_v[0].at[pl.ds(0, 56)],
                            acc.at[pl.ds(s * _STRIPE + 3072, 56)])

        @pl.when(s == _NS - 1)
        def _zero_tail2():
            pltpu.sync_copy(rows_v[0].at[pl.ds(0, 8)],
                            acc.at[pl.ds(s * _STRIPE + 3072, 8)])
        plsc.subcore_barrier()

        def _step(st, carry):
            for k in range(2):
                jp = st * 2 + k
                @pl.when(st > 0)
                def _drain():
                    pltpu.make_async_copy(
                        rows_v[k], acc.at[cidx_v[k]], ssem).wait()
                pltpu.sync_copy(src_t.at[s, jp], src_v[k])
                pltpu.sync_copy(cidx_t.at[s, jp], cidx_v[k])
                pltpu.sync_copy(attr_t.at[s, jp], attr_v[k])
                pltpu.async_copy(htab.at[src_v[k]], rows_v[k], gsem)
            for k in range(2):
                pltpu.make_async_copy(
                    htab.at[src_v[k]], rows_v[k], gsem).wait()
                for g in range(_SCH // 16):
                    a16 = attr_v[k][pl.ds(g * 16, 16)]
                    for i in range(16):
                        e = g * 16 + i
                        spl = _splat16(a16, i)
                        rows_v[k][e, 0:16] = rows_v[k][e, 0:16] * spl
                pltpu.async_copy(
                    rows_v[k], acc.at[cidx_v[k]], ssem, add=True)
            return carry
        lax.fori_loop(0, _PPT // 2, _step, 0)
        for k in range(2):
            pltpu.make_async_copy(
                rows_v[k], acc.at[cidx_v[k]], ssem).wait()
        plsc.subcore_barrier()

        @pl.when(s < _NS - 1)
        def _writeout():
            pltpu.sync_copy(acc.at[pl.ds(s * _STRIPE, _STRIPE)],
                            aq.at[q, pl.ds(s * _STRIPE, _STRIPE)])

        @pl.when(s == _NS - 1)
        def _writeout_tail():
            pltpu.sync_copy(acc.at[pl.ds(s * _STRIPE, _LSTRIPE)],
                            aq.at[q, pl.ds(s * _STRIPE, _LSTRIPE)])
        plsc.subcore_barrier()


def _sc_body_wrap(hq, src_t, cidx_t, attr_t, aq,
                  src_v0, src_v1, cidx_v0, cidx_v1, attr_v0, attr_v1,
                  rows_v0, rows_v1, acc, htab, gsem, ssem):
    _sc_body(hq, src_t, cidx_t, attr_t, aq,
             (src_v0, src_v1), (cidx_v0, cidx_v1), (attr_v0, attr_v1),
             (rows_v0, rows_v1), acc, htab, gsem, ssem)


def _sc_agg(hq, src_t, cidx_t, attr_t):
    kern = pl.kernel(
        _sc_body_wrap,
        out_type=jax.ShapeDtypeStruct((_NQ, _RN, _QW), jnp.float32),
        mesh=plsc.VectorSubcoreMesh(core_axis_name="c", subcore_axis_name="s"),
        compiler_params=pltpu.CompilerParams(use_tc_tiling_on_sc=False),
        scratch_types=[
            pltpu.VMEM((_SCH,), jnp.int32),
            pltpu.VMEM((_SCH,), jnp.int32),
            pltpu.VMEM((_SCH,), jnp.int32),
            pltpu.VMEM((_SCH,), jnp.int32),
            pltpu.VMEM((_SCH,), jnp.float32),
            pltpu.VMEM((_SCH,), jnp.float32),
            pltpu.VMEM((_SCH, _QW), jnp.float32),
            pltpu.VMEM((_SCH, _QW), jnp.float32),
            pltpu.VMEM_SHARED((_RN, _QW), jnp.float32),
            pltpu.VMEM_SHARED((_N, _QW), jnp.float32),
            pltpu.SemaphoreType.DMA,
            pltpu.SemaphoreType.DMA,
        ],
    )
    return kern(hq, src_t, cidx_t, attr_t)


def kernel(x, edge_index, edge_type, edge_attr, W_in, b_in, g_in, be_in,
           Wg, bg, W1, b1, g1, bb1, W2, b2, Wt1, bt1, gt1, bbt1, Wt2, bt2):
    src = edge_index[0]
    dst = edge_index[1]
    cidx = dst * _R + edge_type
    attr = edge_attr[:, 0]

    pad = _EP - _E
    srcp = jnp.pad(src, (0, pad))
    cidxp = jnp.pad(cidx, (0, pad))
    attrp = jnp.pad(attr, (0, pad))

    src_t = srcp.reshape(_NS, _PPT, _SCH)
    cidx_t = cidxp.reshape(_NS, _PPT, _SCH)
    attr_t = attrp.reshape(_NS, _PPT, _SCH)

    h = _proj(x, W_in, b_in, g_in, be_in)

    for l in range(2):
        hq = h.reshape(_N, _NQ, _QW).transpose(1, 0, 2)
        a = _sc_agg(hq, src_t, cidx_t, attr_t)
        a2 = a.reshape(_NQ, _N, _R * _QW)
        wq = jnp.stack([
            Wg[l, :, qq * _QW:(qq + 1) * _QW, :].reshape(_R * _QW, _H)
            for qq in range(_NQ)])
        h = _mid(a2, wq, bg[l])

    return _head(h, W1, b1, g1, bb1, W2, b2, Wt1, bt1, gt1, bbt1, Wt2, bt2)


# speedup vs baseline: 1.4105x; 1.1656x over previous
import functools

import jax
import jax.numpy as jnp
import numpy as np
from jax import lax
from jax.experimental import pallas as pl
from jax.experimental.pallas import tpu as pltpu
from jax.experimental.pallas import tpu_sc as plsc

_N = 10000
_E = 320000
_H = 128
_R = 5
_NQ = 4
_QW = 32
_NS = 16
_NC = 2
_CH = 128
_CPT = 160
_CB = 4
_NB = 2
_EPT = _CH * _CPT
_EP = _NS * _EPT
_RN = _R * _N
_STRIPE = 3128
_LSTRIPE = _RN - 15 * _STRIPE
_BM = 400


def _layer_norm(x, g, b):
    mu = jnp.mean(x, axis=-1, keepdims=True)
    var = jnp.mean(jnp.square(x - mu), axis=-1, keepdims=True)
    return (x - mu) / jnp.sqrt(var + 1e-5) * g + b


def _gelu(x):
    return x * 0.5 * (1.0 + lax.erf(x * np.float32(1.0 / np.sqrt(2.0))))


def _proj_body(x_ref, w_ref, b_ref, g_ref, be_ref, o_ref):
    h = jnp.dot(x_ref[...], w_ref[...], preferred_element_type=jnp.float32)
    o_ref[...] = _gelu(_layer_norm(h + b_ref[...], g_ref[...], be_ref[...]))


def _proj(x, w, b, g, be):
    return pl.pallas_call(
        _proj_body,
        grid=(_N // _BM,),
        in_specs=[
            pl.BlockSpec((_BM, _H), lambda i: (i, 0)),
            pl.BlockSpec((_H, _H), lambda i: (0, 0)),
            pl.BlockSpec((1, _H), lambda i: (0, 0)),
            pl.BlockSpec((1, _H), lambda i: (0, 0)),
            pl.BlockSpec((1, _H), lambda i: (0, 0)),
        ],
        out_specs=pl.BlockSpec((_BM, _H), lambda i: (i, 0)),
        out_shape=jax.ShapeDtypeStruct((_N, _H), jnp.float32),
    )(x, w, b.reshape(1, -1), g.reshape(1, -1), be.reshape(1, -1))


def _mid_body(a_ref, wq_ref, b_ref, o_ref):
    acc = jnp.zeros((_BM, _H), jnp.float32)
    for q in range(_NQ):
        acc += jnp.dot(a_ref[q], wq_ref[q], preferred_element_type=jnp.float32)
    o_ref[...] = _gelu(acc + b_ref[...])


def _mid(a2, wq, b):
    rw = _R * _QW
    return pl.pallas_call(
        _mid_body,
        grid=(_N // _BM,),
        in_specs=[
            pl.BlockSpec((_NQ, _BM, rw), lambda i: (0, i, 0)),
            pl.BlockSpec((_NQ, rw, _H), lambda i: (0, 0, 0)),
            pl.BlockSpec((1, _H), lambda i: (0, 0)),
        ],
        out_specs=pl.BlockSpec((_BM, _H), lambda i: (i, 0)),
        out_shape=jax.ShapeDtypeStruct((_N, _H), jnp.float32),
    )(a2, wq, b.reshape(1, -1))


def _head_body(h_ref, w1_ref, b1_ref, g1_ref, bb1_ref, w2_ref, b2_ref,
               wt1_ref, bt1_ref, gt1_ref, bbt1_ref, wt2_ref, bt2_ref,
               cat_ref, tax_ref):
    h = h_ref[...]
    z = _gelu(_layer_norm(
        jnp.dot(h, w1_ref[...], preferred_element_type=jnp.float32) + b1_ref[...],
        g1_ref[...], bb1_ref[...]))
    cat_ref[...] = jnp.dot(z, w2_ref[...], preferred_element_type=jnp.float32) + b2_ref[...]
    t = _gelu(_layer_norm(
        jnp.dot(h, wt1_ref[...], preferred_element_type=jnp.float32) + bt1_ref[...],
        gt1_ref[...], bbt1_ref[...]))
    tax_ref[...] = jnp.dot(t, wt2_ref[...], preferred_element_type=jnp.float32) + bt2_ref[...]


def _head(h, w1, b1, g1, bb1, w2, b2, wt1, bt1, gt1, bbt1, wt2, bt2):
    h2 = 2 * _H
    out = 400
    tax = 20
    return pl.pallas_call(
        _head_body,
        grid=(_N // _BM,),
        in_specs=[
            pl.BlockSpec((_BM, _H), lambda i: (i, 0)),
            pl.BlockSpec((_H, h2), lambda i: (0, 0)),
            pl.BlockSpec((1, h2), lambda i: (0, 0)),
            pl.BlockSpec((1, h2), lambda i: (0, 0)),
            pl.BlockSpec((1, h2), lambda i: (0, 0)),
            pl.BlockSpec((h2, out), lambda i: (0, 0)),
            pl.BlockSpec((1, out), lambda i: (0, 0)),
            pl.BlockSpec((_H, h2), lambda i: (0, 0)),
            pl.BlockSpec((1, h2), lambda i: (0, 0)),
            pl.BlockSpec((1, h2), lambda i: (0, 0)),
            pl.BlockSpec((1, h2), lambda i: (0, 0)),
            pl.BlockSpec((h2, tax), lambda i: (0, 0)),
            pl.BlockSpec((1, tax), lambda i: (0, 0)),
        ],
        out_specs=[
            pl.BlockSpec((_BM, out), lambda i: (i, 0)),
            pl.BlockSpec((_BM, tax), lambda i: (i, 0)),
        ],
        out_shape=[
            jax.ShapeDtypeStruct((_N, out), jnp.float32),
            jax.ShapeDtypeStruct((_N, tax), jnp.float32),
        ],
    )(h, w1, b1.reshape(1, -1), g1.reshape(1, -1), bb1.reshape(1, -1),
      w2, b2.reshape(1, -1), wt1, bt1.reshape(1, -1), gt1.reshape(1, -1),
      bbt1.reshape(1, -1), wt2, bt2.reshape(1, -1))


def _splat16(v, i):
    return lax.gather(
        v,
        jnp.full((16, 1), i, jnp.int32),
        lax.GatherDimensionNumbers(
            offset_dims=(), collapsed_slice_dims=(0,), start_index_map=(0,)),
        (1,),
        mode=lax.GatherScatterMode.PROMISE_IN_BOUNDS)


def _sc_body(hq, src_t, cidx_t, attr_t, aq,
             src_v, cidx_v, attr_v, rows4, acc, htab, gsem, ssem):
    c = lax.axis_index("c")
    s = lax.axis_index("s")

    for qi in range(2):
        q = 2 * c + qi

        z16 = jnp.zeros((16,), jnp.float32)
        for i in range(_CH):
            rows4[0, i, 0:16] = z16
            rows4[0, i, 16:32] = z16

        @pl.when(s == 0)
        def _load_table():
            pltpu.sync_copy(hq.at[q], htab)

        def _zero(k, carry):
            pltpu.sync_copy(rows4.at[0],
                            acc.at[pl.ds(s * _STRIPE + k * _CH, _CH)])
            return carry
        lax.fori_loop(0, 3072 // _CH, _zero, 0)

        @pl.when(s < _NS - 1)
        def _zero_tail():
            pltpu.sync_copy(rows4.at[0, pl.ds(0, 56)],
                            acc.at[pl.ds(s * _STRIPE + 3072, 56)])

        @pl.when(s == _NS - 1)
        def _zero_tail2():
            pltpu.sync_copy(rows4.at[0, pl.ds(0, 8)],
                            acc.at[pl.ds(s * _STRIPE + 3072, 8)])
        plsc.subcore_barrier()

        def _quad(qq, carry):
            is_stage = qq % (_CB // _NB) == 0

            @pl.when(jnp.logical_and(is_stage, qq > 0))
            def _flush():
                for b in range(_NB):
                    pltpu.make_async_copy(
                        rows4.at[b], acc.at[cidx_v.at[0]], ssem).wait()

            @pl.when(is_stage)
            def _stage():
                jg = qq // (_CB // _NB)
                pltpu.sync_copy(src_t.at[s, pl.ds(jg * _CB, _CB)], src_v)
                pltpu.sync_copy(cidx_t.at[s, pl.ds(jg * _CB, _CB)], cidx_v)
                pltpu.sync_copy(attr_t.at[s, pl.ds(jg * _CB, _CB)], attr_v)

            jr = (qq % (_CB // _NB)) * _NB
            for b in range(_NB):
                @pl.when(jnp.logical_not(is_stage))
                def _drain():
                    pltpu.make_async_copy(
                        rows4.at[b], acc.at[cidx_v.at[0]], ssem).wait()
                pltpu.async_copy(htab.at[src_v.at[jr + b]], rows4.at[b], gsem)
            for b in range(_NB):
                pltpu.make_async_copy(
                    htab.at[src_v.at[jr + b]], rows4.at[b], gsem).wait()
                for g in range(_CH // 16):
                    a16 = attr_v[jr + b, pl.ds(g * 16, 16)]
                    for i in range(16):
                        e = g * 16 + i
                        spl = _splat16(a16, i)
                        rows4[b, e, 0:16] = rows4[b, e, 0:16] * spl
                        rows4[b, e, 16:32] = rows4[b, e, 16:32] * spl
                pltpu.async_copy(
                    rows4.at[b], acc.at[cidx_v.at[jr + b]], ssem, add=True)
            return carry
        lax.fori_loop(0, _CPT // _NB, _quad, 0)
        for b in range(_NB):
            pltpu.make_async_copy(
                rows4.at[b], acc.at[cidx_v.at[0]], ssem).wait()
        plsc.subcore_barrier()

        @pl.when(s < _NS - 1)
        def _writeout():
            pltpu.sync_copy(acc.at[pl.ds(s * _STRIPE, _STRIPE)],
                            aq.at[q, pl.ds(s * _STRIPE, _STRIPE)])

        @pl.when(s == _NS - 1)
        def _writeout_tail():
            pltpu.sync_copy(acc.at[pl.ds(s * _STRIPE, _LSTRIPE)],
                            aq.at[q, pl.ds(s * _STRIPE, _LSTRIPE)])
        plsc.subcore_barrier()


def _sc_agg(hq, src_t, cidx_t, attr_t):
    kern = pl.kernel(
        _sc_body,
        out_type=jax.ShapeDtypeStruct((_NQ, _RN, _QW), jnp.float32),
        mesh=plsc.VectorSubcoreMesh(core_axis_name="c", subcore_axis_name="s"),
        compiler_params=pltpu.CompilerParams(use_tc_tiling_on_sc=False),
        scratch_types=[
            pltpu.VMEM((_CB, _CH), jnp.int32),
            pltpu.VMEM((_CB, _CH), jnp.int32),
            pltpu.VMEM((_CB, _CH), jnp.float32),
            pltpu.VMEM((_NB, _CH, _QW), jnp.float32),
            pltpu.VMEM_SHARED((_RN, _QW), jnp.float32),
            pltpu.VMEM_SHARED((_N, _QW), jnp.float32),
            pltpu.SemaphoreType.DMA,
            pltpu.SemaphoreType.DMA,
        ],
    )
    return kern(hq, src_t, cidx_t, attr_t)


def kernel(x, edge_index, edge_type, edge_attr, W_in, b_in, g_in, be_in,
           Wg, bg, W1, b1, g1, bb1, W2, b2, Wt1, bt1, gt1, bbt1, Wt2, bt2):
    src = edge_index[0]
    dst = edge_index[1]
    cidx = dst * _R + edge_type
    attr = edge_attr[:, 0]

    pad = _EP - _E
    srcp = jnp.pad(src, (0, pad))
    cidxp = jnp.pad(cidx, (0, pad))
    attrp = jnp.pad(attr, (0, pad))

    src_t = srcp.reshape(_NS, _CPT, _CH)
    cidx_t = cidxp.reshape(_NS, _CPT, _CH)
    attr_t = attrp.reshape(_NS, _CPT, _CH)

    h = _proj(x, W_in, b_in, g_in, be_in)

    for l in range(2):
        hq = h.reshape(_N, _NQ, _QW).transpose(1, 0, 2)
        a = _sc_agg(hq, src_t, cidx_t, attr_t)
        a2 = a.reshape(_NQ, _N, _R * _QW)
        wq = jnp.stack([
            Wg[l, :, qq * _QW:(qq + 1) * _QW, :].reshape(_R * _QW, _H)
            for qq in range(_NQ)])
        h = _mid(a2, wq, bg[l])

    return _head(h, W1, b1, g1, bb1, W2, b2, Wt1, bt1, gt1, bbt1, Wt2, bt2)


# speedup vs baseline: 1.6624x; 1.1786x over previous
import functools

import jax
import jax.numpy as jnp
import numpy as np
from jax import lax
from jax.experimental import pallas as pl
from jax.experimental.pallas import tpu as pltpu
from jax.experimental.pallas import tpu_sc as plsc

_N = 10000
_E = 320000
_H = 128
_R = 5
_NQ = 4
_QW = 32
_NS = 16
_NC = 2
_CH = 128
_CPT = 160
_CB = 4
_NB = 2
_EPT = _CH * _CPT
_EP = _NS * _EPT
_RN = _R * _N
_STRIPE = 3128
_LSTRIPE = _RN - 15 * _STRIPE
_BM = 400


def _layer_norm(x, g, b):
    mu = jnp.mean(x, axis=-1, keepdims=True)
    var = jnp.mean(jnp.square(x - mu), axis=-1, keepdims=True)
    return (x - mu) / jnp.sqrt(var + 1e-5) * g + b


def _gelu(x):
    return x * 0.5 * (1.0 + lax.erf(x * np.float32(1.0 / np.sqrt(2.0))))


def _proj_body(x_ref, w_ref, b_ref, g_ref, be_ref, o_ref):
    h = jnp.dot(x_ref[...], w_ref[...], preferred_element_type=jnp.float32)
    o_ref[...] = _gelu(_layer_norm(h + b_ref[...], g_ref[...], be_ref[...]))


def _proj(x, w, b, g, be):
    return pl.pallas_call(
        _proj_body,
        grid=(_N // _BM,),
        in_specs=[
            pl.BlockSpec((_BM, _H), lambda i: (i, 0)),
            pl.BlockSpec((_H, _H), lambda i: (0, 0)),
            pl.BlockSpec((1, _H), lambda i: (0, 0)),
            pl.BlockSpec((1, _H), lambda i: (0, 0)),
            pl.BlockSpec((1, _H), lambda i: (0, 0)),
        ],
        out_specs=pl.BlockSpec((_BM, _H), lambda i: (i, 0)),
        out_shape=jax.ShapeDtypeStruct((_N, _H), jnp.float32),
    )(x, w, b.reshape(1, -1), g.reshape(1, -1), be.reshape(1, -1))


def _mid_body(a_ref, wq_ref, b_ref, o_ref):
    acc = jnp.zeros((_BM, _H), jnp.float32)
    for q in range(_NQ):
        acc += jnp.dot(a_ref[q], wq_ref[q], preferred_element_type=jnp.float32)
    o_ref[...] = _gelu(acc + b_ref[...])


def _mid(a2, wq, b):
    rw = _R * _QW
    return pl.pallas_call(
        _mid_body,
        grid=(_N // _BM,),
        in_specs=[
            pl.BlockSpec((_NQ, _BM, rw), lambda i: (0, i, 0)),
            pl.BlockSpec((_NQ, rw, _H), lambda i: (0, 0, 0)),
            pl.BlockSpec((1, _H), lambda i: (0, 0)),
        ],
        out_specs=pl.BlockSpec((_BM, _H), lambda i: (i, 0)),
        out_shape=jax.ShapeDtypeStruct((_N, _H), jnp.float32),
    )(a2, wq, b.reshape(1, -1))


def _head_body(h_ref, w1_ref, b1_ref, g1_ref, bb1_ref, w2_ref, b2_ref,
               wt1_ref, bt1_ref, gt1_ref, bbt1_ref, wt2_ref, bt2_ref,
               cat_ref, tax_ref):
    h = h_ref[...]
    z = _gelu(_layer_norm(
        jnp.dot(h, w1_ref[...], preferred_element_type=jnp.float32) + b1_ref[...],
        g1_ref[...], bb1_ref[...]))
    cat_ref[...] = jnp.dot(z, w2_ref[...], preferred_element_type=jnp.float32) + b2_ref[...]
    t = _gelu(_layer_norm(
        jnp.dot(h, wt1_ref[...], preferred_element_type=jnp.float32) + bt1_ref[...],
        gt1_ref[...], bbt1_ref[...]))
    tax_ref[...] = jnp.dot(t, wt2_ref[...], preferred_element_type=jnp.float32) + bt2_ref[...]


def _head(h, w1, b1, g1, bb1, w2, b2, wt1, bt1, gt1, bbt1, wt2, bt2):
    h2 = 2 * _H
    out = 400
    tax = 20
    return pl.pallas_call(
        _head_body,
        grid=(_N // _BM,),
        in_specs=[
            pl.BlockSpec((_BM, _H), lambda i: (i, 0)),
            pl.BlockSpec((_H, h2), lambda i: (0, 0)),
            pl.BlockSpec((1, h2), lambda i: (0, 0)),
            pl.BlockSpec((1, h2), lambda i: (0, 0)),
            pl.BlockSpec((1, h2), lambda i: (0, 0)),
            pl.BlockSpec((h2, out), lambda i: (0, 0)),
            pl.BlockSpec((1, out), lambda i: (0, 0)),
            pl.BlockSpec((_H, h2), lambda i: (0, 0)),
            pl.BlockSpec((1, h2), lambda i: (0, 0)),
            pl.BlockSpec((1, h2), lambda i: (0, 0)),
            pl.BlockSpec((1, h2), lambda i: (0, 0)),
            pl.BlockSpec((h2, tax), lambda i: (0, 0)),
            pl.BlockSpec((1, tax), lambda i: (0, 0)),
        ],
        out_specs=[
            pl.BlockSpec((_BM, out), lambda i: (i, 0)),
            pl.BlockSpec((_BM, tax), lambda i: (i, 0)),
        ],
        out_shape=[
            jax.ShapeDtypeStruct((_N, out), jnp.float32),
            jax.ShapeDtypeStruct((_N, tax), jnp.float32),
        ],
    )(h, w1, b1.reshape(1, -1), g1.reshape(1, -1), bb1.reshape(1, -1),
      w2, b2.reshape(1, -1), wt1, bt1.reshape(1, -1), gt1.reshape(1, -1),
      bbt1.reshape(1, -1), wt2, bt2.reshape(1, -1))


def _splat16(v, i):
    return lax.gather(
        v,
        jnp.full((16, 1), i, jnp.int32),
        lax.GatherDimensionNumbers(
            offset_dims=(), collapsed_slice_dims=(0,), start_index_map=(0,)),
        (1,),
        mode=lax.GatherScatterMode.PROMISE_IN_BOUNDS)


def _sc_body(hq, src_t, cidx_t, attr_t, aq,
             src_v, cidx_v, attr_v, rows4, acc, htab, gsem, ssem, isem):
    c = lax.axis_index("c")
    s = lax.axis_index("s")

    for qi in range(2):
        q = 2 * c + qi

        z16 = jnp.zeros((16,), jnp.float32)
        for i in range(_CH):
            rows4[0, i, 0:16] = z16
            rows4[0, i, 16:32] = z16

        @pl.when(s == 0)
        def _load_table():
            pltpu.sync_copy(hq.at[q], htab)

        def _zero(k, carry):
            pltpu.sync_copy(rows4.at[0],
                            acc.at[pl.ds(s * _STRIPE + k * _CH, _CH)])
            return carry
        lax.fori_loop(0, 3072 // _CH, _zero, 0)

        @pl.when(s < _NS - 1)
        def _zero_tail():
            pltpu.sync_copy(rows4.at[0, pl.ds(0, 56)],
                            acc.at[pl.ds(s * _STRIPE + 3072, 56)])

        @pl.when(s == _NS - 1)
        def _zero_tail2():
            pltpu.sync_copy(rows4.at[0, pl.ds(0, 8)],
                            acc.at[pl.ds(s * _STRIPE + 3072, 8)])
        plsc.subcore_barrier()

        pltpu.async_copy(src_t.at[s, pl.ds(0, _NB)],
                         src_v.at[pl.ds(0, _NB)], isem)
        pltpu.async_copy(cidx_t.at[s, pl.ds(0, _NB)],
                         cidx_v.at[pl.ds(0, _NB)], isem)
        pltpu.async_copy(attr_t.at[s, pl.ds(0, _NB)],
                         attr_v.at[pl.ds(0, _NB)], isem)

        def _quad(qq, carry):
            jr = (qq % 2) * _NB
            nr = ((qq + 1) % 2) * _NB

            @pl.when(qq > 0)
            def _drain():
                for b in range(_NB):
                    pltpu.make_async_copy(
                        rows4.at[b], acc.at[cidx_v.at[0]], ssem).wait()

            @pl.when(qq < _CPT // _NB - 1)
            def _prefetch():
                nj = (qq + 1) * _NB
                pltpu.async_copy(src_t.at[s, pl.ds(nj, _NB)],
                                 src_v.at[pl.ds(nr, _NB)], isem)
                pltpu.async_copy(cidx_t.at[s, pl.ds(nj, _NB)],
                                 cidx_v.at[pl.ds(nr, _NB)], isem)
                pltpu.async_copy(attr_t.at[s, pl.ds(nj, _NB)],
                                 attr_v.at[pl.ds(nr, _NB)], isem)

            pltpu.make_async_copy(src_t.at[s, pl.ds(qq * _NB, _NB)],
                                  src_v.at[pl.ds(jr, _NB)], isem).wait()
            pltpu.make_async_copy(cidx_t.at[s, pl.ds(qq * _NB, _NB)],
                                  cidx_v.at[pl.ds(jr, _NB)], isem).wait()
            pltpu.make_async_copy(attr_t.at[s, pl.ds(qq * _NB, _NB)],
                                  attr_v.at[pl.ds(jr, _NB)], isem).wait()

            for b in range(_NB):
                pltpu.async_copy(htab.at[src_v.at[jr + b]], rows4.at[b], gsem)
            for b in range(_NB):
                pltpu.make_async_copy(
                    htab.at[src_v.at[jr + b]], rows4.at[b], gsem).wait()
                for g in range(_CH // 16):
                    a16 = attr_v[jr + b, pl.ds(g * 16, 16)]
                    for i in range(16):
                        e = g * 16 + i
                        spl = _splat16(a16, i)
                        rows4[b, e, 0:16] = rows4[b, e, 0:16] * spl
                        rows4[b, e, 16:32] = rows4[b, e, 16:32] * spl
                pltpu.async_copy(
                    rows4.at[b], acc.at[cidx_v.at[jr + b]], ssem, add=True)
            return carry
        lax.fori_loop(0, _CPT // _NB, _quad, 0)
        for b in range(_NB):
            pltpu.make_async_copy(
                rows4.at[b], acc.at[cidx_v.at[0]], ssem).wait()
        plsc.subcore_barrier()

        @pl.when(s < _NS - 1)
        def _writeout():
            pltpu.sync_copy(acc.at[pl.ds(s * _STRIPE, _STRIPE)],
                            aq.at[q, pl.ds(s * _STRIPE, _STRIPE)])

        @pl.when(s == _NS - 1)
        def _writeout_tail():
            pltpu.sync_copy(acc.at[pl.ds(s * _STRIPE, _LSTRIPE)],
                            aq.at[q, pl.ds(s * _STRIPE, _LSTRIPE)])
        plsc.subcore_barrier()


def _sc_agg(hq, src_t, cidx_t, attr_t):
    kern = pl.kernel(
        _sc_body,
        out_type=jax.ShapeDtypeStruct((_NQ, _RN, _QW), jnp.float32),
        mesh=plsc.VectorSubcoreMesh(core_axis_name="c", subcore_axis_name="s"),
        compiler_params=pltpu.CompilerParams(use_tc_tiling_on_sc=False),
        scratch_types=[
            pltpu.VMEM((_CB, _CH), jnp.int32),
            pltpu.VMEM((_CB, _CH), jnp.int32),
            pltpu.VMEM((_CB, _CH), jnp.float32),
            pltpu.VMEM((_NB, _CH, _QW), jnp.float32),
            pltpu.VMEM_SHARED((_RN, _QW), jnp.float32),
            pltpu.VMEM_SHARED((_N, _QW), jnp.float32),
            pltpu.SemaphoreType.DMA,
            pltpu.SemaphoreType.DMA,
            pltpu.SemaphoreType.DMA,
        ],
    )
    return kern(hq, src_t, cidx_t, attr_t)


def kernel(x, edge_index, edge_type, edge_attr, W_in, b_in, g_in, be_in,
           Wg, bg, W1, b1, g1, bb1, W2, b2, Wt1, bt1, gt1, bbt1, Wt2, bt2):
    src = edge_index[0]
    dst = edge_index[1]
    cidx = dst * _R + edge_type
    attr = edge_attr[:, 0]

    pad = _EP - _E
    srcp = jnp.pad(src, (0, pad))
    cidxp = jnp.pad(cidx, (0, pad))
    attrp = jnp.pad(attr, (0, pad))

    src_t = srcp.reshape(_NS, _CPT, _CH)
    cidx_t = cidxp.reshape(_NS, _CPT, _CH)
    attr_t = attrp.reshape(_NS, _CPT, _CH)

    h = _proj(x, W_in, b_in, g_in, be_in)

    for l in range(2):
        hq = h.reshape(_N, _NQ, _QW).transpose(1, 0, 2)
        a = _sc_agg(hq, src_t, cidx_t, attr_t)
        a2 = a.reshape(_NQ, _N, _R * _QW)
        wq = jnp.stack([
            Wg[l, :, qq * _QW:(qq + 1) * _QW, :].reshape(_R * _QW, _H)
            for qq in range(_NQ)])
        h = _mid(a2, wq, bg[l])

    return _head(h, W1, b1, g1, bb1, W2, b2, Wt1, bt1, gt1, bbt1, Wt2, bt2)
